# Initial kernel scaffold; baseline (speedup 1.0000x reference)
import jax
import jax.numpy as jnp
from jax.experimental import pallas as pl


def kernel(x, table):
    raise NotImplementedError("write your pallas kernel here")


# speedup vs baseline: 1.3579x; 1.3579x over previous
import jax
import jax.numpy as jnp
from jax import lax
from jax.experimental import pallas as pl
from jax.experimental.pallas import tpu as pltpu
from jax.experimental.pallas import tpu_sc as plsc

NUM_EMBEDDINGS = 1000000
DIM = 32
LANES = 16
NUM_WORKERS = 32
ROW = 128
ROWS_PER_CHUNK = 8
TOTAL = 16384 * 26
ROWS_TOTAL = TOTAL // ROW
ROWS_PER_W = ROWS_TOTAL // NUM_WORKERS
N_CHUNKS = ROWS_PER_W // ROWS_PER_CHUNK


def _emb_body(idx_hbm, table_hbm, out_hbm, idx_v, rows_a, rows_b, sem_a, sem_b):
    wid = lax.axis_index("s") * 2 + lax.axis_index("c")
    r0 = wid * ROWS_PER_W

    pltpu.sync_copy(idx_hbm.at[pl.ds(r0, ROWS_PER_W)], idx_v)

    def fix_row(r, carry):
        for k in range(ROW // LANES):
            v = idx_v[r, pl.ds(k * LANES, LANES)]
            idx_v[r, pl.ds(k * LANES, LANES)] = jnp.minimum(
                jnp.maximum(v - 1, 0), NUM_EMBEDDINGS - 1)
        return carry

    lax.fori_loop(0, ROWS_PER_W, fix_row, 0)

    def fire_chunk(g, buf, sem):
        for j in range(ROWS_PER_CHUNK):
            pltpu.async_copy(
                table_hbm.at[idx_v.at[g * ROWS_PER_CHUNK + j]], buf.at[j], sem)

    def wait_chunk(buf, sem):
        pltpu.make_async_copy(
            out_hbm.at[pl.ds(0, ROWS_PER_CHUNK)], buf, sem).wait()

    def wb_chunk(g, buf):
        pltpu.sync_copy(
            buf, out_hbm.at[pl.ds(r0 + g * ROWS_PER_CHUNK, ROWS_PER_CHUNK)])

    fire_chunk(0, rows_a, sem_a)

    def pair(p, carry):
        g = 2 * p
        fire_chunk(g + 1, rows_b, sem_b)
        wait_chunk(rows_a, sem_a)
        wb_chunk(g, rows_a)
        fire_chunk(g + 2, rows_a, sem_a)
        wait_chunk(rows_b, sem_b)
        wb_chunk(g + 1, rows_b)
        return carry

    lax.fori_loop(0, (N_CHUNKS - 1) // 2, pair, 0)
    wait_chunk(rows_a, sem_a)
    wb_chunk(N_CHUNKS - 1, rows_a)


_emb_call = pl.kernel(
    _emb_body,
    out_type=jax.ShapeDtypeStruct((ROWS_TOTAL, ROW, DIM), jnp.float32),
    mesh=plsc.VectorSubcoreMesh(core_axis_name="c", subcore_axis_name="s"),
    compiler_params=pltpu.CompilerParams(use_tc_tiling_on_sc=False),
    scratch_types=[
        pltpu.VMEM((ROWS_PER_W, ROW), jnp.int32),
        pltpu.VMEM((ROWS_PER_CHUNK, ROW, DIM), jnp.float32),
        pltpu.VMEM((ROWS_PER_CHUNK, ROW, DIM), jnp.float32),
        pltpu.SemaphoreType.DMA,
        pltpu.SemaphoreType.DMA,
    ],
)


@jax.jit
def kernel(x, table):
    idx2d = x.reshape(ROWS_TOTAL, ROW)
    out = _emb_call(idx2d, table)
    return out.reshape(x.shape[0], x.shape[1], DIM)
